# baseline (speedup 1.0000x reference)
import jax
import jax.numpy as jnp
from jax import lax
from jax.experimental import pallas as pl
from jax.experimental.pallas import tpu as pltpu

NUM_BITS = 8
MIN_VAL = 0.0
MAX_VAL = 1.0
THRESHOLD = 3.0
B = 1024
F = 512
HIDDEN = 2048
CLASSES = 1000
CLP = 1024
ENC = F * NUM_BITS

BB = 256
HB = 256
HC = 256

NB = B // BB
NH = HIDDEN // HB
NCL = HIDDEN // HC
S_Z0 = NB
S_Z1 = S_Z0 + NH // 2
S_OUT = S_Z1 + NH // 2
N_OUT = NCL
N_STEPS = S_OUT + N_OUT


def _body(x_ref, w0a_ref, w0b_ref, w1a_ref, w1b_ref, oca_ref, ocb_ref,
          out_ref, pred_ref, enc_s, a0_s, a1_s, r_s):
    i = pl.program_id(0)

    @pl.when(i == 0)
    def _():
        src = lax.broadcasted_iota(jnp.int32, (F, ENC), 1) // NUM_BITS
        dst = lax.broadcasted_iota(jnp.int32, (F, ENC), 0)
        r_s[...] = (src == dst).astype(jnp.bfloat16)

    @pl.when(i < S_Z0)
    def _():
        xb = x_ref[...]
        xc = jnp.clip(xb, MIN_VAL, MAX_VAL)
        norm = (xc - MIN_VAL) / (MAX_VAL - MIN_VAL)
        lv = jnp.round(norm * (2 ** NUM_BITS - 1)).astype(jnp.int32)
        gray = lv ^ (lv >> 1)
        rep = lax.dot_general(gray.astype(jnp.bfloat16), r_s[...],
                              (((1,), (0,)), ((), ())),
                              preferred_element_type=jnp.float32)
        gi = rep.astype(jnp.int32)
        kidx = lax.broadcasted_iota(jnp.int32, (BB, ENC), 1) & (NUM_BITS - 1)
        enc_s[pl.ds(i * BB, BB), :] = ((gi >> kidx) & 1).astype(jnp.bfloat16)

    def layer_step(step0, act_s, wa_ref, wb_ref, dst_s):
        h2 = (i - step0) * 2
        for h, wref in ((h2, wa_ref), (h2 + 1, wb_ref)):
            wb = wref[...].astype(jnp.bfloat16)
            z = lax.dot_general(act_s[...], wb, (((1,), (1,)), ((), ())),
                                preferred_element_type=jnp.float32)
            dst_s[:, pl.ds(h * HB, HB)] = (z >= THRESHOLD).astype(jnp.bfloat16)

    @pl.when((i >= S_Z0) & (i < S_Z1))
    def _():
        layer_step(S_Z0, enc_s, w0a_ref, w0b_ref, a0_s)

    @pl.when((i >= S_Z1) & (i < S_OUT))
    def _():
        layer_step(S_Z1, a0_s, w1a_ref, w1b_ref, a1_s)

    @pl.when(i >= S_OUT)
    def _():
        j = i - S_OUT
        hba = 2 * lax.rem(j, N_OUT // 2)

        def acc(a_s):
            p = lax.dot_general(
                a_s[:, pl.ds(hba * HC, HC)].astype(jnp.float32), oca_ref[0],
                (((1,), (0,)), ((), ())), preferred_element_type=jnp.float32)
            return p + lax.dot_general(
                a_s[:, pl.ds((hba + 1) * HC, HC)].astype(jnp.float32),
                ocb_ref[0],
                (((1,), (0,)), ((), ())), preferred_element_type=jnp.float32)

        @pl.when(j == 0)
        def _():
            out_ref[...] = acc(a0_s)

        @pl.when((j > 0) & (j < N_OUT // 2))
        def _():
            out_ref[...] = out_ref[...] + acc(a0_s)

        @pl.when(j >= N_OUT // 2)
        def _():
            out_ref[...] = out_ref[...] + acc(a1_s)

        @pl.when(j == N_OUT - 1)
        def _():
            out = out_ref[...]
            mx = jnp.max(out, axis=1, keepdims=True)
            idx = lax.broadcasted_iota(jnp.int32, out.shape, 1)
            pred = jnp.min(jnp.where(out == mx, idx, CLP), axis=1)
            pred_ref[...] = pred.reshape(NB, 1, BB).astype(jnp.int32)


def kernel(trainOrTest, x, y, W0, W1, outC):
    del trainOrTest, y
    ocp = jnp.pad(outC, ((0, 0), (0, 0), (0, CLP - CLASSES)))

    def w_index(step0, off):
        def f(i):
            return (jnp.clip(i - step0, 0, NH // 2 - 1) * 2 + off, 0)
        return f

    def oc_index(off):
        def f(i):
            j = jnp.clip(i - S_OUT, 0, N_OUT - 1)
            return (j // (N_OUT // 2), 2 * (j % (N_OUT // 2)) + off, 0)
        return f

    out_act, preds3 = pl.pallas_call(
        _body,
        grid=(N_STEPS,),
        in_specs=[
            pl.BlockSpec((BB, F), lambda i: (jnp.minimum(i, NB - 1), 0)),
            pl.BlockSpec((HB, ENC), w_index(S_Z0, 0)),
            pl.BlockSpec((HB, ENC), w_index(S_Z0, 1)),
            pl.BlockSpec((HB, HIDDEN), w_index(S_Z1, 0)),
            pl.BlockSpec((HB, HIDDEN), w_index(S_Z1, 1)),
            pl.BlockSpec((1, HC, CLP), oc_index(0)),
            pl.BlockSpec((1, HC, CLP), oc_index(1)),
        ],
        out_specs=[
            pl.BlockSpec((B, CLP), lambda i: (0, 0)),
            pl.BlockSpec((NB, 1, BB), lambda i: (0, 0, 0)),
        ],
        out_shape=[
            jax.ShapeDtypeStruct((B, CLP), jnp.float32),
            jax.ShapeDtypeStruct((NB, 1, BB), jnp.int32),
        ],
        scratch_shapes=[
            pltpu.VMEM((B, ENC), jnp.bfloat16),
            pltpu.VMEM((B, HIDDEN), jnp.bfloat16),
            pltpu.VMEM((B, HIDDEN), jnp.bfloat16),
            pltpu.VMEM((F, ENC), jnp.bfloat16),
        ],
    )(x, W0, W0, W1, W1, ocp, ocp)

    predictions = preds3.reshape(B)
    return predictions, out_act[:, :CLASSES]


# speedup vs baseline: 1.3395x; 1.3395x over previous
import jax
import jax.numpy as jnp
from jax import lax
from jax.experimental import pallas as pl
from jax.experimental.pallas import tpu as pltpu

NUM_BITS = 8
MIN_VAL = 0.0
MAX_VAL = 1.0
THRESHOLD = 3.0
B = 1024
F = 512
HIDDEN = 2048
CLASSES = 1000
ENC = F * NUM_BITS

BB = 256
HB = 256
HC = 256

NB = B // BB
NH = HIDDEN // HB
NCL = HIDDEN // HC
S_Z0 = NB
S_Z1 = S_Z0 + NH // 2
S_OUT = S_Z1 + NH // 2
N_OUT = NCL
N_STEPS = S_OUT + N_OUT


def _body(x_ref, w0a_ref, w0b_ref, w1a_ref, w1b_ref, oca_ref, ocb_ref,
          out_ref, pred_ref, enc_s, a0_s, a1_s, r_s):
    i = pl.program_id(0)

    @pl.when(i == 0)
    def _():
        src = lax.broadcasted_iota(jnp.int32, (F, ENC), 1) // NUM_BITS
        dst = lax.broadcasted_iota(jnp.int32, (F, ENC), 0)
        r_s[...] = (src == dst).astype(jnp.bfloat16)

    @pl.when(i < S_Z0)
    def _():
        xb = x_ref[...]
        xc = jnp.clip(xb, MIN_VAL, MAX_VAL)
        norm = (xc - MIN_VAL) / (MAX_VAL - MIN_VAL)
        lv = jnp.round(norm * (2 ** NUM_BITS - 1)).astype(jnp.int32)
        gray = lv ^ (lv >> 1)
        rep = lax.dot_general(gray.astype(jnp.bfloat16), r_s[...],
                              (((1,), (0,)), ((), ())),
                              preferred_element_type=jnp.float32)
        gi = rep.astype(jnp.int32)
        kidx = lax.broadcasted_iota(jnp.int32, (BB, ENC), 1) & (NUM_BITS - 1)
        enc_s[pl.ds(i * BB, BB), :] = ((gi >> kidx) & 1).astype(jnp.bfloat16)

    def layer_step(step0, act_s, wa_ref, wb_ref, dst_s):
        h2 = (i - step0) * 2
        for h, wref in ((h2, wa_ref), (h2 + 1, wb_ref)):
            wb = wref[...].astype(jnp.bfloat16)
            z = lax.dot_general(act_s[...], wb, (((1,), (1,)), ((), ())),
                                preferred_element_type=jnp.float32)
            dst_s[:, pl.ds(h * HB, HB)] = (z >= THRESHOLD).astype(jnp.bfloat16)

    @pl.when((i >= S_Z0) & (i < S_Z1))
    def _():
        layer_step(S_Z0, enc_s, w0a_ref, w0b_ref, a0_s)

    @pl.when((i >= S_Z1) & (i < S_OUT))
    def _():
        layer_step(S_Z1, a0_s, w1a_ref, w1b_ref, a1_s)

    @pl.when(i >= S_OUT)
    def _():
        j = i - S_OUT
        hba = 2 * lax.rem(j, N_OUT // 2)

        def acc(a_s):
            p = lax.dot_general(
                a_s[:, pl.ds(hba * HC, HC)].astype(jnp.float32), oca_ref[0],
                (((1,), (0,)), ((), ())), preferred_element_type=jnp.float32)
            return p + lax.dot_general(
                a_s[:, pl.ds((hba + 1) * HC, HC)].astype(jnp.float32),
                ocb_ref[0],
                (((1,), (0,)), ((), ())), preferred_element_type=jnp.float32)

        @pl.when(j == 0)
        def _():
            out_ref[...] = acc(a0_s)

        @pl.when((j > 0) & (j < N_OUT // 2))
        def _():
            out_ref[...] = out_ref[...] + acc(a0_s)

        @pl.when(j >= N_OUT // 2)
        def _():
            out_ref[...] = out_ref[...] + acc(a1_s)

        @pl.when(j == N_OUT - 1)
        def _():
            out = out_ref[...]
            mx = jnp.max(out, axis=1, keepdims=True)
            idx = lax.broadcasted_iota(jnp.int32, out.shape, 1)
            pred = jnp.min(jnp.where(out == mx, idx, CLASSES), axis=1)
            pred_ref[...] = pred.reshape(NB, 1, BB).astype(jnp.int32)


def kernel(trainOrTest, x, y, W0, W1, outC):
    del trainOrTest, y

    def w_index(step0, off):
        def f(i):
            return (jnp.clip(i - step0, 0, NH // 2 - 1) * 2 + off, 0)
        return f

    def oc_index(off):
        def f(i):
            j = jnp.clip(i - S_OUT, 0, N_OUT - 1)
            return (j // (N_OUT // 2), 2 * (j % (N_OUT // 2)) + off, 0)
        return f

    out_act, preds3 = pl.pallas_call(
        _body,
        grid=(N_STEPS,),
        in_specs=[
            pl.BlockSpec((BB, F), lambda i: (jnp.minimum(i, NB - 1), 0)),
            pl.BlockSpec((HB, ENC), w_index(S_Z0, 0)),
            pl.BlockSpec((HB, ENC), w_index(S_Z0, 1)),
            pl.BlockSpec((HB, HIDDEN), w_index(S_Z1, 0)),
            pl.BlockSpec((HB, HIDDEN), w_index(S_Z1, 1)),
            pl.BlockSpec((1, HC, CLASSES), oc_index(0)),
            pl.BlockSpec((1, HC, CLASSES), oc_index(1)),
        ],
        out_specs=[
            pl.BlockSpec((B, CLASSES), lambda i: (0, 0)),
            pl.BlockSpec((NB, 1, BB), lambda i: (0, 0, 0)),
        ],
        out_shape=[
            jax.ShapeDtypeStruct((B, CLASSES), jnp.float32),
            jax.ShapeDtypeStruct((NB, 1, BB), jnp.int32),
        ],
        scratch_shapes=[
            pltpu.VMEM((B, ENC), jnp.bfloat16),
            pltpu.VMEM((B, HIDDEN), jnp.bfloat16),
            pltpu.VMEM((B, HIDDEN), jnp.bfloat16),
            pltpu.VMEM((F, ENC), jnp.bfloat16),
        ],
    )(x, W0, W0, W1, W1, outC, outC)

    predictions = preds3.reshape(B)
    return predictions, out_act
